# baseline (speedup 1.0000x reference)
import jax
import jax.numpy as jnp
from jax.experimental import pallas as pl
from jax.experimental.pallas import tpu as pltpu

_K = 8


def _body(dist_ref, w_ref, out_ref, acc_ref):
    k = pl.program_id(0)
    x = dist_ref[0:1, :]
    y = dist_ref[1:2, :]
    w = w_ref[...]

    @pl.when(k == 0)
    def _init():
        acc_ref[...] = jnp.zeros_like(acc_ref)

    s0 = jnp.sum(w, axis=1, keepdims=True)
    s1x = jnp.sum(w * x, axis=1, keepdims=True)
    s1y = jnp.sum(w * y, axis=1, keepdims=True)
    t2 = jnp.sum(w * (x * x + y * y), keepdims=True)
    acc_ref[:, 0:1] += s0
    acc_ref[:, 1:2] += s1x
    acc_ref[:, 2:3] += s1y
    acc_ref[0:1, 3:4] += t2

    @pl.when(k == _K - 1)
    def _finalize():
        s0f = acc_ref[:, 0:1]
        s1xf = acc_ref[:, 1:2]
        s1yf = acc_ref[:, 2:3]
        quad = (s1xf * s1xf + s1yf * s1yf) * (s0f - 2.0)
        total = jnp.sum(quad, axis=0, keepdims=True) + acc_ref[0:1, 3:4]
        out_ref[...] = total * (0.5 / acc_ref.shape[0])


def kernel(distribution, weights, spot_dist):
    del spot_dist
    n = weights.shape[1]
    nb = n // _K
    dist_t = distribution.T
    out = pl.pallas_call(
        _body,
        grid=(_K,),
        in_specs=[
            pl.BlockSpec((2, nb), lambda k: (0, k)),
            pl.BlockSpec((weights.shape[0], nb), lambda k: (0, k)),
        ],
        out_specs=pl.BlockSpec((1, 1), lambda k: (0, 0)),
        out_shape=jax.ShapeDtypeStruct((1, 1), jnp.float32),
        scratch_shapes=[pltpu.VMEM((weights.shape[0], 8), jnp.float32)],
    )(dist_t, weights)
    return out[0, 0]
